# Initial kernel scaffold; baseline (speedup 1.0000x reference)
import jax
import jax.numpy as jnp
from jax.experimental import pallas as pl


def kernel(user_emb, item_emb, e, v, adj_val, adj_row, adj_col):
    raise NotImplementedError("write your pallas kernel here")


# speedup vs baseline: 3.0655x; 3.0655x over previous
import dataclasses
import functools

import jax
import jax.numpy as jnp
from jax import lax
from jax.experimental import pallas as pl
from jax.experimental.pallas import tpu as pltpu
from jax.experimental.pallas import tpu_sc as plsc

N = 50000
H = N // 2
HP = 25088
D = 64
E = 800000
RANK = 128

NC, NS = 2, 16
W = 256
SUB = 128
NSUB = W // SUB
NWIN = 196
EPT = NWIN * W
E_PAD = EPT * NS
ROWS_PT = HP // NS

RB = 2000
NB = N // RB


def _sc_compiler_params():
    cp = pltpu.CompilerParams()
    fields = pltpu.CompilerParams.__dataclass_fields__
    if "needs_layout_passes" in fields:
        cp = dataclasses.replace(cp, needs_layout_passes=False)
    if "use_tc_tiling_on_sc" in fields:
        cp = dataclasses.replace(cp, use_tc_tiling_on_sc=False)
    return cp


def _spmm_body(x_hbm, col_hbm, dst_hbm, val_hbm, out_hbm,
               cidx, didx, vbuf, rows, acc, sem):
    c = lax.axis_index("c")
    s = lax.axis_index("s")

    @pl.loop(0, W)
    def _(i):
        for j4 in range(D // 16):
            rows[i, pl.ds(j4 * 16, 16)] = jnp.zeros((16,), jnp.float32)

    nfull, tail = ROWS_PT // W, ROWS_PT % W
    for k in range(nfull):
        pltpu.sync_copy(rows, acc.at[pl.ds(s * ROWS_PT + k * W, W)])
    if tail:
        pltpu.sync_copy(rows.at[pl.ds(0, tail)],
                        acc.at[pl.ds(s * ROWS_PT + nfull * W, tail)])
    plsc.subcore_barrier()

    @pl.loop(0, NWIN)
    def _(w):
        ebase = s * EPT + w * W
        rbase = s * (EPT // SUB) + w * NSUB
        pltpu.sync_copy(col_hbm.at[pl.ds(rbase, NSUB)], cidx)
        pltpu.sync_copy(dst_hbm.at[pl.ds(rbase, NSUB)], didx)
        pltpu.sync_copy(val_hbm.at[c, pl.ds(ebase, W)], vbuf)
        descs = [
            pltpu.async_copy(x_hbm.at[cidx.at[j]],
                             rows.at[pl.ds(j * SUB, SUB)], sem)
            for j in range(NSUB)
        ]
        for dsc in descs:
            dsc.wait()

        @pl.loop(0, W)
        def _(ei):
            vv = plsc.load_gather(vbuf, [jnp.full((16,), ei, jnp.int32)])
            for j4 in range(D // 16):
                sl = (ei, pl.ds(j4 * 16, 16))
                rows[sl] = rows[sl] * vv

        for j in range(NSUB):
            pltpu.sync_copy(rows.at[pl.ds(j * SUB, SUB)],
                            acc.at[didx.at[j]], add=True)

    plsc.subcore_barrier()
    pltpu.sync_copy(acc.at[pl.ds(s * ROWS_PT, ROWS_PT)],
                    out_hbm.at[c, pl.ds(s * ROWS_PT, ROWS_PT)])


def _spmm(x, col2d, dst2d, val2):
    k = pl.kernel(
        _spmm_body,
        out_type=jax.ShapeDtypeStruct((NC, HP, D), jnp.float32),
        mesh=plsc.VectorSubcoreMesh(core_axis_name="c", subcore_axis_name="s",
                                    num_cores=NC, num_subcores=NS),
        scratch_types=[
            pltpu.VMEM((NSUB, SUB), jnp.int32),
            pltpu.VMEM((NSUB, SUB), jnp.int32),
            pltpu.VMEM((W,), jnp.float32),
            pltpu.VMEM((W, D), jnp.float32),
            pltpu.VMEM_SHARED((HP, D), jnp.float32),
            pltpu.SemaphoreType.DMA,
        ],
        compiler_params=_sc_compiler_params(),
    )
    return k(x, col2d, dst2d, val2)


def _lowrank_body(e_ref, v_ref, x_ref, low1_ref, low2_ref, p1, g, p2):
    p = pl.program_id(0)
    i = pl.program_id(1)
    hi = jax.lax.Precision.HIGHEST
    eb = e_ref[...]
    ev = eb * v_ref[...]

    @pl.when(jnp.logical_and(p == 0, i == 0))
    def _():
        p1[...] = jnp.zeros_like(p1)
        g[...] = jnp.zeros_like(g)

    @pl.when(p == 0)
    def _():
        xb = x_ref[...]
        p1[...] += lax.dot_general(eb, xb, (((0,), (0,)), ((), ())),
                                   precision=hi,
                                   preferred_element_type=jnp.float32)
        g[...] += lax.dot_general(eb, ev, (((0,), (0,)), ((), ())),
                                  precision=hi,
                                  preferred_element_type=jnp.float32)

    @pl.when(jnp.logical_and(p == 1, i == 0))
    def _():
        p2[...] = jnp.dot(g[...], p1[...], precision=hi,
                          preferred_element_type=jnp.float32)

    @pl.when(p == 1)
    def _():
        low1_ref[...] = jnp.dot(ev, p1[...], precision=hi,
                                preferred_element_type=jnp.float32)
        low2_ref[...] = jnp.dot(ev, p2[...], precision=hi,
                                preferred_element_type=jnp.float32)


def _lowrank(e, v, x0):
    out_sds = jax.ShapeDtypeStruct((N, D), jnp.float32)
    return pl.pallas_call(
        _lowrank_body,
        grid=(2, NB),
        in_specs=[
            pl.BlockSpec((RB, RANK), lambda p, i: (i, 0)),
            pl.BlockSpec((1, RANK), lambda p, i: (0, 0)),
            pl.BlockSpec((RB, D), lambda p, i: (i, 0)),
        ],
        out_specs=[
            pl.BlockSpec((RB, D), lambda p, i: (i, 0)),
            pl.BlockSpec((RB, D), lambda p, i: (i, 0)),
        ],
        out_shape=[out_sds, out_sds],
        scratch_shapes=[
            pltpu.VMEM((RANK, D), jnp.float32),
            pltpu.VMEM((RANK, RANK), jnp.float32),
            pltpu.VMEM((RANK, D), jnp.float32),
        ],
    )(e, v.reshape(1, RANK), x0)


def kernel(user_emb, item_emb, e, v, adj_val, adj_row, adj_col):
    x0 = jnp.concatenate([user_emb, item_emb], axis=0)
    row = adj_row.astype(jnp.int32)
    col = adj_col.astype(jnp.int32)
    val = adj_val.astype(jnp.float32)

    pad = E_PAD - E
    rowp = jnp.concatenate([row, jnp.zeros((pad,), jnp.int32)])
    colp = jnp.concatenate([col, jnp.zeros((pad,), jnp.int32)])
    valp = jnp.concatenate([val, jnp.zeros((pad,), jnp.float32)])

    dst2d = (rowp % H).reshape(-1, SUB)
    in0 = rowp < H
    val2 = jnp.stack([jnp.where(in0, valp, 0.0),
                      jnp.where(in0, 0.0, valp)])
    col_l1 = colp.reshape(-1, SUB)
    col_l2 = (colp + (colp >= H).astype(jnp.int32) * (HP - H)).reshape(-1, SUB)

    ego1p = _spmm(x0, col_l1, dst2d, val2)
    ego2p = _spmm(ego1p.reshape(NC * HP, D), col_l2, dst2d, val2)
    ego1 = jnp.concatenate([ego1p[0, :H], ego1p[1, :H]], axis=0)
    ego2 = jnp.concatenate([ego2p[0, :H], ego2p[1, :H]], axis=0)

    low1, low2 = _lowrank(e, v, x0)

    all_emb = jnp.stack([x0, ego1, ego2], axis=0)
    all_low = jnp.stack([x0, low1, low2], axis=0)
    return (all_emb, all_low)


# speedup vs baseline: 6.5269x; 2.1291x over previous
import dataclasses
import functools

import jax
import jax.numpy as jnp
from jax import lax
from jax.experimental import pallas as pl
from jax.experimental.pallas import tpu as pltpu
from jax.experimental.pallas import tpu_sc as plsc

N = 50000
D = 64
DH = 32
HPALL = 50048
E = 800000
RANK = 128

NC, NS = 2, 16
W = 128
EPT = E // NS
NWIN = EPT // W
UNROLL = 6
OUTER = NWIN // UNROLL
TAIL = EPT - NWIN * W
NROW = 6
NIDX = 6
SDRAIN = 4
ROWS_PT = HPALL // NS

RB = 2000
NB = N // RB


def _sc_compiler_params():
    cp = pltpu.CompilerParams()
    fields = pltpu.CompilerParams.__dataclass_fields__
    if "needs_layout_passes" in fields:
        cp = dataclasses.replace(cp, needs_layout_passes=False)
    if "use_tc_tiling_on_sc" in fields:
        cp = dataclasses.replace(cp, use_tc_tiling_on_sc=False)
    return cp


def _make_spmm_body(xh):
    def body(x_hbm, col_hbm, dst_hbm, val_hbm, out_hbm, *scr):
        return _spmm_impl(xh, x_hbm, col_hbm, dst_hbm, val_hbm, out_hbm, *scr)
    return body


def _spmm_impl(xh, x_hbm, col_hbm, dst_hbm, val_hbm, out_hbm, *scr):
    o = 0
    rows = scr[o:o + NROW]; o += NROW
    cidx = scr[o:o + NIDX]; o += NIDX
    didx = scr[o:o + NIDX]; o += NIDX
    vbuf = scr[o:o + NIDX]; o += NIDX
    acc = scr[o]; o += 1
    gsem = scr[o:o + NROW]; o += NROW
    ssem = scr[o:o + NROW]; o += NROW
    isem = scr[o:o + NIDX]; o += NIDX
    tcol, tdst, tval, trows = scr[o:o + 4]

    c = lax.axis_index("c")
    s = lax.axis_index("s")
    coff = c * xh

    @pl.loop(0, W)
    def _(i):
        for j4 in range(DH // 16):
            rows[0][i, pl.ds(j4 * 16, 16)] = jnp.zeros((16,), jnp.float32)

    nfull, tail = ROWS_PT // W, ROWS_PT % W
    for k in range(nfull):
        pltpu.sync_copy(rows[0], acc.at[pl.ds(s * ROWS_PT + k * W, W)])
    if tail:
        pltpu.sync_copy(rows[0].at[pl.ds(0, tail)],
                        acc.at[pl.ds(s * ROWS_PT + nfull * W, tail)])
    plsc.subcore_barrier()

    def fire_idx(w, q):
        base = s * EPT + w * W
        pltpu.async_copy(col_hbm.at[pl.ds(base, W)], cidx[q], isem[q])
        pltpu.async_copy(dst_hbm.at[pl.ds(base, W)], didx[q], isem[q])
        pltpu.async_copy(val_hbm.at[pl.ds(base, W)], vbuf[q], isem[q])

    def wait_idx(w, q):
        base = s * EPT + w * W
        pltpu.make_async_copy(col_hbm.at[pl.ds(base, W)], cidx[q],
                              isem[q]).wait()
        pltpu.make_async_copy(dst_hbm.at[pl.ds(base, W)], didx[q],
                              isem[q]).wait()
        pltpu.make_async_copy(val_hbm.at[pl.ds(base, W)], vbuf[q],
                              isem[q]).wait()
        for j in range(W // 16):
            sl = pl.ds(j * 16, 16)
            cidx[q][sl] = cidx[q][sl] + coff

    def fire_gather(p, q):
        pltpu.async_copy(x_hbm.at[cidx[q]], rows[p], gsem[p])

    def wait_gather(p, q):
        pltpu.make_async_copy(x_hbm.at[cidx[q]], rows[p], gsem[p]).wait()

    def fire_scatter(p, q):
        pltpu.async_copy(rows[p], acc.at[didx[q]], ssem[p], add=True)

    def wait_scatter(p, q):
        pltpu.make_async_copy(rows[p], acc.at[didx[q]], ssem[p]).wait()

    fire_idx(0, 0)
    fire_idx(1, 1)
    wait_idx(0, 0)
    fire_gather(0, 0)

    @pl.loop(0, OUTER)
    def _(outer):
        for k in range(UNROLL):
            w = outer * UNROLL + k
            p, q = k % NROW, k % NIDX
            pn1, qn1 = (k + 1) % NROW, (k + 1) % NIDX
            qn2 = (k + 2) % NIDX

            if k >= SDRAIN:
                wait_scatter((k - SDRAIN) % NROW, (k - SDRAIN) % NIDX)
            else:
                @pl.when(outer > 0)
                def _():
                    wait_scatter((k - SDRAIN) % NROW, (k - SDRAIN) % NIDX)
            if k < UNROLL - 1:
                wait_idx(w + 1, qn1)
                fire_gather(pn1, qn1)
            else:
                @pl.when(outer < OUTER - 1)
                def _():
                    wait_idx(w + 1, qn1)
                    fire_gather(pn1, qn1)
            if k < UNROLL - 2:
                fire_idx(w + 2, qn2)
            else:
                @pl.when(outer < OUTER - 1)
                def _():
                    fire_idx(w + 2, qn2)
            wait_gather(p, q)

            @pl.loop(0, W)
            def _(ei):
                vv = plsc.load_gather(vbuf[q],
                                      [jnp.full((16,), ei, jnp.int32)])
                for j4 in range(DH // 16):
                    sl = (ei, pl.ds(j4 * 16, 16))
                    rows[p][sl] = rows[p][sl] * vv

            fire_scatter(p, q)

    for j in range(SDRAIN, 0, -1):
        wait_scatter((NWIN - j) % NROW, (NWIN - j) % NIDX)

    tbase = s * EPT + NWIN * W
    pltpu.sync_copy(col_hbm.at[pl.ds(tbase, TAIL)], tcol)
    pltpu.sync_copy(dst_hbm.at[pl.ds(tbase, TAIL)], tdst)
    pltpu.sync_copy(val_hbm.at[pl.ds(tbase, TAIL)], tval)
    for j in range(TAIL // 16):
        sl = pl.ds(j * 16, 16)
        tcol[sl] = tcol[sl] + coff
    pltpu.sync_copy(x_hbm.at[tcol], trows)

    @pl.loop(0, TAIL)
    def _(ei):
        vv = plsc.load_gather(tval, [jnp.full((16,), ei, jnp.int32)])
        for j4 in range(DH // 16):
            sl = (ei, pl.ds(j4 * 16, 16))
            trows[sl] = trows[sl] * vv

    pltpu.sync_copy(trows, acc.at[tdst], add=True)

    plsc.subcore_barrier()
    pltpu.sync_copy(acc.at[pl.ds(s * ROWS_PT, ROWS_PT)],
                    out_hbm.at[c, pl.ds(s * ROWS_PT, ROWS_PT)])


def _spmm(x, col, dst, val, xh):
    k = pl.kernel(
        _make_spmm_body(xh),
        out_type=jax.ShapeDtypeStruct((NC, HPALL, DH), jnp.float32),
        mesh=plsc.VectorSubcoreMesh(core_axis_name="c", subcore_axis_name="s",
                                    num_cores=NC, num_subcores=NS),
        scratch_types=(
            [pltpu.VMEM((W, DH), jnp.float32)] * NROW
            + [pltpu.VMEM((W,), jnp.int32)] * NIDX
            + [pltpu.VMEM((W,), jnp.int32)] * NIDX
            + [pltpu.VMEM((W,), jnp.float32)] * NIDX
            + [pltpu.VMEM_SHARED((HPALL, DH), jnp.float32)]
            + [pltpu.SemaphoreType.DMA] * (2 * NROW + NIDX)
            + [pltpu.VMEM((TAIL,), jnp.int32),
               pltpu.VMEM((TAIL,), jnp.int32),
               pltpu.VMEM((TAIL,), jnp.float32),
               pltpu.VMEM((TAIL, DH), jnp.float32)]
        ),
        compiler_params=_sc_compiler_params(),
    )
    return k(x, col, dst, val)


def _split_bf16(x):
    head = x.astype(jnp.bfloat16)
    tail = (x - head.astype(jnp.float32)).astype(jnp.bfloat16)
    return head, tail


def _dot3(x, y, dims):
    xh, xt = _split_bf16(x)
    yh, yt = _split_bf16(y)
    dn = (dims, ((), ()))
    f32 = jnp.float32
    return (lax.dot_general(xh, yh, dn, preferred_element_type=f32)
            + lax.dot_general(xh, yt, dn, preferred_element_type=f32)
            + lax.dot_general(xt, yh, dn, preferred_element_type=f32))


def _lowrank_body(e_ref, v_ref, x_ref, low1_ref, low2_ref, p1, g, p2):
    p = pl.program_id(0)
    i = pl.program_id(1)
    eb = e_ref[...]
    ev = eb * v_ref[...]

    @pl.when(jnp.logical_and(p == 0, i == 0))
    def _():
        p1[...] = jnp.zeros_like(p1)
        g[...] = jnp.zeros_like(g)

    @pl.when(p == 0)
    def _():
        xb = x_ref[...]
        p1[...] += _dot3(eb, xb, ((0,), (0,)))
        g[...] += _dot3(eb, ev, ((0,), (0,)))

    @pl.when(jnp.logical_and(p == 1, i == 0))
    def _():
        p2[...] = _dot3(g[...], p1[...], ((1,), (0,)))

    @pl.when(p == 1)
    def _():
        low1_ref[...] = _dot3(ev, p1[...], ((1,), (0,)))
        low2_ref[...] = _dot3(ev, p2[...], ((1,), (0,)))


def _lowrank(e, v, x0):
    out_sds = jax.ShapeDtypeStruct((N, D), jnp.float32)
    return pl.pallas_call(
        _lowrank_body,
        grid=(2, NB),
        in_specs=[
            pl.BlockSpec((RB, RANK), lambda p, i: (i, 0)),
            pl.BlockSpec((1, RANK), lambda p, i: (0, 0)),
            pl.BlockSpec((RB, D), lambda p, i: (i, 0)),
        ],
        out_specs=[
            pl.BlockSpec((RB, D), lambda p, i: (i, 0)),
            pl.BlockSpec((RB, D), lambda p, i: (i, 0)),
        ],
        out_shape=[out_sds, out_sds],
        scratch_shapes=[
            pltpu.VMEM((RANK, D), jnp.float32),
            pltpu.VMEM((RANK, RANK), jnp.float32),
            pltpu.VMEM((RANK, D), jnp.float32),
        ],
    )(e, v.reshape(1, RANK), x0)


def kernel(user_emb, item_emb, e, v, adj_val, adj_row, adj_col):
    x0 = jnp.concatenate([user_emb, item_emb], axis=0)
    row = adj_row.astype(jnp.int32)
    col = adj_col.astype(jnp.int32)
    val = adj_val.astype(jnp.float32)

    x0s = jnp.concatenate([user_emb[:, :DH], item_emb[:, :DH],
                           user_emb[:, DH:], item_emb[:, DH:]],
                          axis=0)
    ego1p = _spmm(x0s, col, row, val, N)
    ego2p = _spmm(ego1p.reshape(NC * HPALL, DH), col, row, val, HPALL)
    ego1 = jnp.concatenate([ego1p[0, :N], ego1p[1, :N]], axis=1)
    ego2 = jnp.concatenate([ego2p[0, :N], ego2p[1, :N]], axis=1)

    low1, low2 = _lowrank(e, v, x0)

    all_emb = jnp.stack([x0, ego1, ego2], axis=0)
    all_low = jnp.stack([x0, low1, low2], axis=0)
    return (all_emb, all_low)
